# baseline (speedup 1.0000x reference)
import functools

import jax
import jax.numpy as jnp
from jax import lax
from jax.experimental import pallas as pl
from jax.experimental.pallas import tpu as pltpu
from jax.experimental.pallas import tpu_sc as plsc

_B = 4096
_D = 4096
_NC = 2
_NS = 16
_NW = _NC * _NS
_RPW = _B // _NW
_W = 8
_NCH = _RPW // _W
_L = 16


def kernel(x, pe):
    xf = x.astype(jnp.float32)
    mesh = plsc.VectorSubcoreMesh(core_axis_name="c", subcore_axis_name="s")

    @functools.partial(
        pl.kernel,
        mesh=mesh,
        out_type=jax.ShapeDtypeStruct((_B, _D), jnp.float32),
        scratch_types=[
            pltpu.VMEM((_D,), jnp.float32),
            pltpu.VMEM((_W,), jnp.int32),
            pltpu.VMEM((_W, _D), jnp.float32),
            pltpu.SemaphoreType.DMA,
        ],
    )
    def emb_kernel(x_hbm, xf_hbm, pe_hbm, out_hbm, xf_v, idx_v, rows_v, sem):
        wid = lax.axis_index("s") * _NC + lax.axis_index("c")
        base = wid * _RPW
        pltpu.sync_copy(xf_hbm, xf_v)

        @pl.loop(0, _NCH)
        def _chunk(ch):
            row0 = base + ch * _W
            pltpu.sync_copy(x_hbm.at[pl.ds(row0, _W)], idx_v)
            pltpu.async_copy(pe_hbm.at[idx_v], rows_v, sem).wait()

            @pl.loop(0, _W)
            def _row(r):
                @pl.loop(0, _D, step=_L)
                def _col(cc):
                    rows_v.at[r, pl.ds(cc, _L)][...] = (
                        rows_v.at[r, pl.ds(cc, _L)][...]
                        + xf_v.at[pl.ds(cc, _L)][...]
                    )

            pltpu.sync_copy(rows_v, out_hbm.at[pl.ds(row0, _W)])

    return emb_kernel(x, xf, pe)


# speedup vs baseline: 3.1334x; 3.1334x over previous
import functools

import jax
import jax.numpy as jnp
from jax import lax
from jax.experimental import pallas as pl
from jax.experimental.pallas import tpu as pltpu
from jax.experimental.pallas import tpu_sc as plsc

_B = 4096
_D = 4096
_NC = 2
_NS = 16
_NW = _NC * _NS
_RPW = _B // _NW
_W = 8
_NITEMS = _RPW // _W
_NBUF = 3
_L = 16


def kernel(x, pe):
    xf = x.astype(jnp.float32)
    mesh = plsc.VectorSubcoreMesh(core_axis_name="c", subcore_axis_name="s")

    @functools.partial(
        pl.kernel,
        mesh=mesh,
        out_type=jax.ShapeDtypeStruct((_B, _D), jnp.float32),
        scratch_types=[
            pltpu.VMEM((_D,), jnp.float32),
            pltpu.VMEM((_RPW,), jnp.int32),
            pltpu.VMEM((_NBUF, _W, _D), jnp.float32),
            pltpu.SemaphoreType.DMA,
            pltpu.SemaphoreType.DMA,
            pltpu.SemaphoreType.DMA,
            pltpu.SemaphoreType.DMA,
            pltpu.SemaphoreType.DMA,
            pltpu.SemaphoreType.DMA,
        ],
    )
    def emb_kernel(x_hbm, xf_hbm, pe_hbm, out_hbm, xf_v, idx_v, rows_v,
                   g_sem0, g_sem1, g_sem2, o_sem0, o_sem1, o_sem2):
        wid = lax.axis_index("s") * _NC + lax.axis_index("c")
        base = wid * _RPW
        g_sems = (g_sem0, g_sem1, g_sem2)
        o_sems = (o_sem0, o_sem1, o_sem2)

        pltpu.sync_copy(x_hbm.at[pl.ds(base, _RPW)], idx_v)
        pltpu.sync_copy(xf_hbm, xf_v)

        def start_gather(item):
            b = item % _NBUF
            return pltpu.async_copy(
                pe_hbm.at[idx_v.at[pl.ds(item * _W, _W)]],
                rows_v.at[b], g_sems[b])

        def add_rows(b):
            buf = rows_v.at[b]

            @pl.loop(0, _D, step=_L)
            def _col(cc):
                xv = xf_v[pl.ds(cc, _L)]
                for r in range(_W):
                    buf.at[r, pl.ds(cc, _L)][...] = (
                        buf.at[r, pl.ds(cc, _L)][...] + xv)

        gathers = {}
        outs = {}
        for g in range(min(2, _NITEMS)):
            gathers[g] = start_gather(g)

        for g in range(_NITEMS):
            b = g % _NBUF
            gathers[g].wait()
            add_rows(b)
            outs[g] = pltpu.async_copy(
                rows_v.at[b], out_hbm.at[pl.ds(base + g * _W, _W)], o_sems[b])
            nxt = g + 2
            if nxt < _NITEMS:
                if nxt >= _NBUF:
                    outs[nxt - _NBUF].wait()
                gathers[nxt] = start_gather(nxt)

        for g in range(_NITEMS - _NBUF, _NITEMS):
            outs[g].wait()

    return emb_kernel(x, xf, pe)


# speedup vs baseline: 3.5940x; 1.1470x over previous
import functools

import jax
import jax.numpy as jnp
from jax import lax
from jax.experimental import pallas as pl
from jax.experimental.pallas import tpu as pltpu
from jax.experimental.pallas import tpu_sc as plsc

_B = 4096
_D = 4096
_NC = 2
_NS = 16
_NW = _NC * _NS
_RPW = _B // _NW
_W = 8
_NITEMS = _RPW // _W
_NBUF = 3
_L = 16
_CU = 2


def kernel(x, pe):
    mesh = plsc.VectorSubcoreMesh(core_axis_name="c", subcore_axis_name="s")

    @functools.partial(
        pl.kernel,
        mesh=mesh,
        out_type=jax.ShapeDtypeStruct((_B, _D), jnp.float32),
        scratch_types=[
            pltpu.VMEM((_B,), jnp.int32),
            pltpu.VMEM((_D,), jnp.float32),
            pltpu.VMEM((_NBUF, _W, _D), jnp.float32),
            pltpu.SemaphoreType.DMA,
            pltpu.SemaphoreType.DMA,
            pltpu.SemaphoreType.DMA,
            pltpu.SemaphoreType.DMA,
            pltpu.SemaphoreType.DMA,
            pltpu.SemaphoreType.DMA,
        ],
    )
    def emb_kernel(x_hbm, pe_hbm, out_hbm, xi_v, xf_v, rows_v,
                   g_sem0, g_sem1, g_sem2, o_sem0, o_sem1, o_sem2):
        wid = lax.axis_index("s") * _NC + lax.axis_index("c")
        base = wid * _RPW
        g_sems = (g_sem0, g_sem1, g_sem2)
        o_sems = (o_sem0, o_sem1, o_sem2)

        pltpu.sync_copy(x_hbm, xi_v)

        @pl.loop(0, _D, step=_L)
        def _cvt(cc):
            xf_v[pl.ds(cc, _L)] = lax.convert_element_type(
                xi_v[pl.ds(cc, _L)], jnp.float32)

        def start_gather(item):
            b = item % _NBUF
            return pltpu.async_copy(
                pe_hbm.at[xi_v.at[pl.ds(base + item * _W, _W)]],
                rows_v.at[b], g_sems[b])

        def add_rows(b):
            buf = rows_v.at[b]

            @pl.loop(0, _D, step=_CU * _L)
            def _col(cc):
                for u in range(_CU):
                    xv = xf_v[pl.ds(cc + u * _L, _L)]
                    for r in range(_W):
                        plsc.addupdate(buf.at[r, pl.ds(cc + u * _L, _L)], xv)

        gathers = {}
        outs = {}
        for g in range(min(2, _NITEMS)):
            gathers[g] = start_gather(g)

        for g in range(_NITEMS):
            b = g % _NBUF
            gathers[g].wait()
            add_rows(b)
            outs[g] = pltpu.async_copy(
                rows_v.at[b], out_hbm.at[pl.ds(base + g * _W, _W)], o_sems[b])
            nxt = g + 2
            if nxt < _NITEMS:
                if nxt >= _NBUF:
                    outs[nxt - _NBUF].wait()
                gathers[nxt] = start_gather(nxt)

        for g in range(_NITEMS - _NBUF, _NITEMS):
            outs[g].wait()

    return emb_kernel(x, pe)


# speedup vs baseline: 3.5959x; 1.0005x over previous
import functools

import jax
import jax.numpy as jnp
from jax import lax
from jax.experimental import pallas as pl
from jax.experimental.pallas import tpu as pltpu
from jax.experimental.pallas import tpu_sc as plsc

_B = 4096
_D = 4096
_NC = 2
_NS = 16
_NW = _NC * _NS
_RPW = _B // _NW
_W = 8
_NITEMS = _RPW // _W
_NBUF = 3
_L = 16
_CU = 4


def kernel(x, pe):
    mesh = plsc.VectorSubcoreMesh(core_axis_name="c", subcore_axis_name="s")

    @functools.partial(
        pl.kernel,
        mesh=mesh,
        out_type=jax.ShapeDtypeStruct((_B, _D), jnp.float32),
        scratch_types=[
            pltpu.VMEM((_B,), jnp.int32),
            pltpu.VMEM((_D,), jnp.float32),
            pltpu.VMEM((_NBUF, _W, _D), jnp.float32),
            pltpu.SemaphoreType.DMA,
            pltpu.SemaphoreType.DMA,
            pltpu.SemaphoreType.DMA,
            pltpu.SemaphoreType.DMA,
            pltpu.SemaphoreType.DMA,
            pltpu.SemaphoreType.DMA,
        ],
    )
    def emb_kernel(x_hbm, pe_hbm, out_hbm, xi_v, xf_v, rows_v,
                   g_sem0, g_sem1, g_sem2, o_sem0, o_sem1, o_sem2):
        wid = lax.axis_index("s") * _NC + lax.axis_index("c")
        base = wid * _RPW
        g_sems = (g_sem0, g_sem1, g_sem2)
        o_sems = (o_sem0, o_sem1, o_sem2)

        pltpu.sync_copy(x_hbm, xi_v)

        @plsc.parallel_loop(0, _D, step=_L, unroll=4)
        def _cvt(cc):
            xf_v[pl.ds(cc, _L)] = lax.convert_element_type(
                xi_v[pl.ds(cc, _L)], jnp.float32)

        def start_gather(item):
            b = item % _NBUF
            return pltpu.async_copy(
                pe_hbm.at[xi_v.at[pl.ds(base + item * _W, _W)]],
                rows_v.at[b], g_sems[b])

        def add_rows(b):
            buf = rows_v.at[b]

            @plsc.parallel_loop(0, _D, step=_L, unroll=_CU)
            def _col(cc):
                xv = xf_v[pl.ds(cc, _L)]
                for r in range(_W):
                    plsc.addupdate(buf.at[r, pl.ds(cc, _L)], xv)

        gathers = {}
        outs = {}
        for g in range(min(2, _NITEMS)):
            gathers[g] = start_gather(g)

        for g in range(_NITEMS):
            b = g % _NBUF
            gathers[g].wait()
            add_rows(b)
            outs[g] = pltpu.async_copy(
                rows_v.at[b], out_hbm.at[pl.ds(base + g * _W, _W)], o_sems[b])
            nxt = g + 2
            if nxt < _NITEMS:
                if nxt >= _NBUF:
                    outs[nxt - _NBUF].wait()
                gathers[nxt] = start_gather(nxt)

        for g in range(_NITEMS - _NBUF, _NITEMS):
            outs[g].wait()

    return emb_kernel(x, pe)


# speedup vs baseline: 3.6485x; 1.0146x over previous
import functools

import jax
import jax.numpy as jnp
from jax import lax
from jax.experimental import pallas as pl
from jax.experimental.pallas import tpu as pltpu
from jax.experimental.pallas import tpu_sc as plsc

_B = 4096
_D = 4096
_NC = 2
_NS = 16
_NW = _NC * _NS
_RPW = _B // _NW
_W = 8
_NITEMS = _RPW // _W
_NBUF = 3
_L = 16
_CU = 4


def kernel(x, pe):
    mesh = plsc.VectorSubcoreMesh(core_axis_name="c", subcore_axis_name="s")

    @functools.partial(
        pl.kernel,
        mesh=mesh,
        out_type=jax.ShapeDtypeStruct((_B, _D), jnp.float32),
        scratch_types=[
            pltpu.VMEM((_B,), jnp.int32),
            pltpu.VMEM((_D,), jnp.float32),
            pltpu.VMEM((_NBUF, _W, _D), jnp.float32),
            pltpu.SemaphoreType.DMA,
            pltpu.SemaphoreType.DMA,
            pltpu.SemaphoreType.DMA,
            pltpu.SemaphoreType.DMA,
            pltpu.SemaphoreType.DMA,
            pltpu.SemaphoreType.DMA,
        ],
    )
    def emb_kernel(x_hbm, pe_hbm, out_hbm, xi_v, xf_v, rows_v,
                   g_sem0, g_sem1, g_sem2, o_sem0, o_sem1, o_sem2):
        wid = lax.axis_index("s") * _NC + lax.axis_index("c")
        base = wid * _RPW
        g_sems = (g_sem0, g_sem1, g_sem2)
        o_sems = (o_sem0, o_sem1, o_sem2)

        pltpu.sync_copy(x_hbm, xi_v)

        @plsc.parallel_loop(0, _D, step=_L, unroll=4)
        def _cvt(cc):
            xf_v[pl.ds(cc, _L)] = lax.convert_element_type(
                xi_v[pl.ds(cc, _L)], jnp.float32)

        def start_gather(item):
            b = item % _NBUF
            return pltpu.async_copy(
                pe_hbm.at[xi_v.at[pl.ds(base + item * _W, _W)]],
                rows_v.at[b], g_sems[b])

        def add_rows(b, r0, nr):
            buf = rows_v.at[b]

            @plsc.parallel_loop(0, _D, step=_L, unroll=_CU)
            def _col(cc):
                xv = xf_v[pl.ds(cc, _L)]
                for r in range(r0, r0 + nr):
                    plsc.addupdate(buf.at[r, pl.ds(cc, _L)], xv)

        gathers = {}
        outs = {}
        for g in range(min(2, _NITEMS)):
            gathers[g] = start_gather(g)

        _H = _W // 2
        for g in range(_NITEMS):
            b = g % _NBUF
            gathers[g].wait()
            nxt = g + 2
            if nxt < _NITEMS:
                if nxt >= _NBUF:
                    outs[nxt - _NBUF][0].wait()
                    outs[nxt - _NBUF][1].wait()
                gathers[nxt] = start_gather(nxt)
            add_rows(b, 0, _H)
            o_first = pltpu.async_copy(
                rows_v.at[b].at[pl.ds(0, _H)],
                out_hbm.at[pl.ds(base + g * _W, _H)], o_sems[b])
            add_rows(b, _H, _H)
            o_second = pltpu.async_copy(
                rows_v.at[b].at[pl.ds(_H, _H)],
                out_hbm.at[pl.ds(base + g * _W + _H, _H)], o_sems[b])
            outs[g] = (o_first, o_second)

        for g in range(_NITEMS - _NBUF, _NITEMS):
            outs[g][0].wait()
            outs[g][1].wait()

    return emb_kernel(x, pe)


# speedup vs baseline: 3.6810x; 1.0089x over previous
import functools

import jax
import jax.numpy as jnp
from jax import lax
from jax.experimental import pallas as pl
from jax.experimental.pallas import tpu as pltpu
from jax.experimental.pallas import tpu_sc as plsc

_B = 4096
_D = 4096
_NC = 2
_NS = 16
_NW = _NC * _NS
_RPW = _B // _NW
_W = 8
_NITEMS = _RPW // _W
_NBUF = 3
_L = 16
_CU = 4


def kernel(x, pe):
    mesh = plsc.VectorSubcoreMesh(core_axis_name="c", subcore_axis_name="s")

    @functools.partial(
        pl.kernel,
        mesh=mesh,
        out_type=jax.ShapeDtypeStruct((_B, _D), jnp.float32),
        scratch_types=[
            pltpu.VMEM((_B,), jnp.int32),
            pltpu.VMEM((_D,), jnp.float32),
            pltpu.VMEM((_NBUF, _W, _D), jnp.float32),
            pltpu.SemaphoreType.DMA,
            pltpu.SemaphoreType.DMA,
            pltpu.SemaphoreType.DMA,
            pltpu.SemaphoreType.DMA,
            pltpu.SemaphoreType.DMA,
            pltpu.SemaphoreType.DMA,
        ],
    )
    def emb_kernel(x_hbm, pe_hbm, out_hbm, xi_v, xf_v, rows_v,
                   g_sem0, g_sem1, g_sem2, o_sem0, o_sem1, o_sem2):
        wid = lax.axis_index("s") * _NC + lax.axis_index("c")
        base = wid * _RPW
        g_sems = (g_sem0, g_sem1, g_sem2)
        o_sems = (o_sem0, o_sem1, o_sem2)

        pltpu.sync_copy(x_hbm, xi_v)

        def start_gather(item):
            b = item % _NBUF
            return pltpu.async_copy(
                pe_hbm.at[xi_v.at[pl.ds(base + item * _W, _W)]],
                rows_v.at[b], g_sems[b])

        def add_rows(b, r0, nr):
            buf = rows_v.at[b]

            @plsc.parallel_loop(0, _D, step=_L, unroll=_CU)
            def _col(cc):
                xv = xf_v[pl.ds(cc, _L)]
                for r in range(r0, r0 + nr):
                    plsc.addupdate(buf.at[r, pl.ds(cc, _L)], xv)

        gathers = {}
        outs = {}
        for g in range(min(2, _NITEMS)):
            gathers[g] = start_gather(g)

        @plsc.parallel_loop(0, _D, step=_L, unroll=4)
        def _cvt(cc):
            xf_v[pl.ds(cc, _L)] = lax.convert_element_type(
                xi_v[pl.ds(cc, _L)], jnp.float32)

        _H = _W // 2
        for g in range(_NITEMS):
            b = g % _NBUF
            gathers[g].wait()
            nxt = g + 2
            if nxt < _NITEMS:
                if nxt >= _NBUF:
                    outs[nxt - _NBUF][0].wait()
                    outs[nxt - _NBUF][1].wait()
                gathers[nxt] = start_gather(nxt)
            add_rows(b, 0, _H)
            o_first = pltpu.async_copy(
                rows_v.at[b].at[pl.ds(0, _H)],
                out_hbm.at[pl.ds(base + g * _W, _H)], o_sems[b])
            add_rows(b, _H, _H)
            o_second = pltpu.async_copy(
                rows_v.at[b].at[pl.ds(_H, _H)],
                out_hbm.at[pl.ds(base + g * _W + _H, _H)], o_sems[b])
            outs[g] = (o_first, o_second)

        for g in range(_NITEMS - _NBUF, _NITEMS):
            outs[g][0].wait()
            outs[g][1].wait()

    return emb_kernel(x, pe)
